# baseline (speedup 1.0000x reference)
import functools

import jax
import jax.numpy as jnp
from jax import lax
from jax.experimental import pallas as pl
from jax.experimental.pallas import tpu as pltpu
from jax.experimental.pallas import tpu_sc as plsc

D = 64
CHUNK = 256
NB = 4
EPS = 1e-5
LN_BATCH = 16


@functools.lru_cache(maxsize=None)
def _build_gather(n_tok):
    info = plsc.get_sparse_core_info()
    nc, ns = info.num_cores, info.num_subcores
    nw = nc * ns
    per_w = n_tok // nw
    n_l = per_w // CHUNK
    assert n_tok == nw * per_w and per_w == n_l * CHUNK and n_l % NB == 0
    mesh = plsc.VectorSubcoreMesh(core_axis_name="c", subcore_axis_name="s")

    def body(idx_hbm, table_hbm, out_hbm, idx_v, rows_v, sem_g, sem_s):
        wid = lax.axis_index("s") * nc + lax.axis_index("c")
        pltpu.sync_copy(idx_hbm.at[pl.ds(wid * n_l, n_l)], idx_v)
        r_base = wid * (per_w // 2)

        def gather_cp(l, b, h):
            return pltpu.make_async_copy(
                table_hbm.at[idx_v.at[l, h]], rows_v.at[b, h], sem_g.at[b, h])

        def store_cp(l, b, h):
            r0 = r_base + l * (CHUNK // 2)
            return pltpu.make_async_copy(
                rows_v.at[b, h],
                out_hbm.at[pl.ds(r0, CHUNK // 2), pl.ds(h * D, D)],
                sem_s.at[b, h])

        for b in range(NB - 1):
            gather_cp(b, b, 0).start()
            gather_cp(b, b, 1).start()

        def loop_body(i, carry):
            for b in range(NB):
                l = NB * i + b
                gather_cp(l, b, 0).wait()
                gather_cp(l, b, 1).wait()
                store_cp(l, b, 0).start()
                store_cp(l, b, 1).start()
                ln = l + NB - 1
                bn = (b + NB - 1) % NB
                if b == 0:
                    @pl.when(i >= 1)
                    def _():
                        store_cp(l - 1, bn, 0).wait()
                        store_cp(l - 1, bn, 1).wait()
                    gather_cp(ln, bn, 0).start()
                    gather_cp(ln, bn, 1).start()
                else:
                    @pl.when(i < n_l // NB - 1)
                    def _():
                        store_cp(l - 1, bn, 0).wait()
                        store_cp(l - 1, bn, 1).wait()
                        gather_cp(ln, bn, 0).start()
                        gather_cp(ln, bn, 1).start()
            return carry

        lax.fori_loop(0, n_l // NB, loop_body, 0)
        for b in range(NB):
            store_cp(n_l - NB + b, b, 0).wait()
            store_cp(n_l - NB + b, b, 1).wait()

    return pl.kernel(
        body,
        out_type=jax.ShapeDtypeStruct((n_tok // 2, 2 * D), jnp.float32),
        mesh=mesh,
        compiler_params=pltpu.CompilerParams(
            needs_layout_passes=False, use_tc_tiling_on_sc=False
        ),
        scratch_types=[
            pltpu.VMEM((n_l, 2, CHUNK // 2), jnp.int32),
            pltpu.VMEM((NB, 2, CHUNK // 2, D), jnp.float32),
            pltpu.SemaphoreType.DMA((NB, 2)),
            pltpu.SemaphoreType.DMA((NB, 2)),
        ],
    )


def _ln_body(g2_ref, b2_ref, x_ref, o_ref):
    x = x_ref[...]
    xa = x[:, :D]
    xb = x[:, D:]
    ma = jnp.sum(xa, axis=1, keepdims=True) * (1.0 / D)
    mb = jnp.sum(xb, axis=1, keepdims=True) * (1.0 / D)
    va = jnp.sum(xa * xa, axis=1, keepdims=True) * (1.0 / D) - ma * ma
    vb = jnp.sum(xb * xb, axis=1, keepdims=True) * (1.0 / D) - mb * mb
    ia = lax.rsqrt(va + EPS)
    ib = lax.rsqrt(vb + EPS)
    n = x.shape[0]
    scale = jnp.concatenate(
        [jnp.broadcast_to(ia, (n, D)), jnp.broadcast_to(ib, (n, D))], axis=1)
    shift = jnp.concatenate(
        [jnp.broadcast_to(ma, (n, D)), jnp.broadcast_to(mb, (n, D))], axis=1)
    o_ref[...] = (x - shift) * scale * g2_ref[...] + b2_ref[...]


@functools.lru_cache(maxsize=None)
def _build_ln(n_b, n_l):
    bb = LN_BATCH
    rows = bb * n_l // 2
    assert n_b % bb == 0 and n_l % 2 == 0
    return pl.pallas_call(
        _ln_body,
        grid=(n_b // bb,),
        in_specs=[
            pl.BlockSpec((1, 2 * D), lambda i: (0, 0)),
            pl.BlockSpec((1, 2 * D), lambda i: (0, 0)),
            pl.BlockSpec((rows, 2 * D), lambda i: (i, 0)),
        ],
        out_specs=pl.BlockSpec((rows, 2 * D), lambda i: (i, 0)),
        out_shape=jax.ShapeDtypeStruct((n_b * n_l // 2, 2 * D), jnp.float32),
    )


def kernel(x, table, gamma, beta):
    n_b, n_l = x.shape
    idx = x.reshape(-1)
    if idx.dtype != jnp.int32:
        idx = idx.astype(jnp.int32)
    idx2 = idx.reshape(-1, CHUNK // 2, 2).transpose(0, 2, 1)
    gathered = _build_gather(idx.size)(idx2, table)
    g2 = jnp.tile(gamma, 2).reshape(1, 2 * D)
    b2 = jnp.tile(beta, 2).reshape(1, 2 * D)
    y = _build_ln(n_b, n_l)(g2, b2, gathered)
    return y.reshape(n_b, n_l, D)


# speedup vs baseline: 1.0874x; 1.0874x over previous
import functools

import jax
import jax.numpy as jnp
from jax import lax
from jax.experimental import pallas as pl
from jax.experimental.pallas import tpu as pltpu
from jax.experimental.pallas import tpu_sc as plsc

D = 64
CHUNK = 128
NB = 4
EPS = 1e-5
LN_ROWS = 4096


@functools.lru_cache(maxsize=None)
def _build_gather(n_tok):
    info = plsc.get_sparse_core_info()
    nc, ns = info.num_cores, info.num_subcores
    nw = nc * ns
    per_w = n_tok // nw
    n_l = per_w // CHUNK
    assert n_tok == nw * per_w and per_w == n_l * CHUNK and n_l % NB == 0
    mesh = plsc.VectorSubcoreMesh(core_axis_name="c", subcore_axis_name="s")

    def body(idx_hbm, table_hbm, out_hbm, idx_v, rows_v, sem_g, sem_s):
        wid = lax.axis_index("s") * nc + lax.axis_index("c")
        pltpu.sync_copy(idx_hbm.at[pl.ds(wid * n_l, n_l)], idx_v)
        t_base = wid * per_w

        def gather_cp(l, b):
            return pltpu.make_async_copy(
                table_hbm.at[idx_v.at[l]], rows_v.at[b], sem_g.at[b])

        def store_cp(l, b):
            return pltpu.make_async_copy(
                rows_v.at[b],
                out_hbm.at[pl.ds(t_base + l * CHUNK, CHUNK)],
                sem_s.at[b])

        for b in range(NB - 1):
            gather_cp(b, b).start()

        def loop_body(i, carry):
            for b in range(NB):
                l = NB * i + b
                gather_cp(l, b).wait()
                store_cp(l, b).start()
                ln = l + NB - 1
                bn = (b + NB - 1) % NB
                if b == 0:
                    @pl.when(i >= 1)
                    def _():
                        store_cp(l - 1, bn).wait()
                    gather_cp(ln, bn).start()
                else:
                    @pl.when(i < n_l // NB - 1)
                    def _():
                        store_cp(l - 1, bn).wait()
                        gather_cp(ln, bn).start()
            return carry

        lax.fori_loop(0, n_l // NB, loop_body, 0)
        for b in range(NB):
            store_cp(n_l - NB + b, b).wait()

    return pl.kernel(
        body,
        out_type=jax.ShapeDtypeStruct((n_tok, D), jnp.float32),
        mesh=mesh,
        compiler_params=pltpu.CompilerParams(
            needs_layout_passes=False, use_tc_tiling_on_sc=False
        ),
        scratch_types=[
            pltpu.VMEM((n_l, CHUNK), jnp.int32),
            pltpu.VMEM((NB, CHUNK, D), jnp.float32),
            pltpu.SemaphoreType.DMA((NB,)),
            pltpu.SemaphoreType.DMA((NB,)),
        ],
    )


def _ln_body(g_ref, b_ref, x_ref, o_ref):
    x = x_ref[...]
    m = jnp.sum(x, axis=1, keepdims=True) * (1.0 / D)
    v = jnp.sum(x * x, axis=1, keepdims=True) * (1.0 / D) - m * m
    inv = lax.rsqrt(v + EPS)
    o_ref[...] = (x - m) * inv * g_ref[...] + b_ref[...]


@functools.lru_cache(maxsize=None)
def _build_ln(n_tok):
    assert n_tok % LN_ROWS == 0
    return pl.pallas_call(
        _ln_body,
        grid=(n_tok // LN_ROWS,),
        in_specs=[
            pl.BlockSpec((1, D), lambda i: (0, 0)),
            pl.BlockSpec((1, D), lambda i: (0, 0)),
            pl.BlockSpec((LN_ROWS, D), lambda i: (i, 0)),
        ],
        out_specs=pl.BlockSpec((LN_ROWS, D), lambda i: (i, 0)),
        out_shape=jax.ShapeDtypeStruct((n_tok, D), jnp.float32),
    )


def kernel(x, table, gamma, beta):
    n_b, n_l = x.shape
    idx = x.reshape(-1)
    if idx.dtype != jnp.int32:
        idx = idx.astype(jnp.int32)
    idx2 = idx.reshape(-1, CHUNK)
    gathered = _build_gather(idx.size)(idx2, table)
    y = _build_ln(idx.size)(gamma.reshape(1, D), beta.reshape(1, D), gathered)
    return y.reshape(n_b, n_l, D)
